# initial kernel scaffold (unmeasured)
import jax
import jax.numpy as jnp
from jax import lax
from jax.experimental import pallas as pl
from jax.experimental.pallas import tpu as pltpu

N_DEV = 4
M = 6144
EPS = 1e-5

TM = 512
TS = 256
NC = M // TM
NS = M // TS


def _eye(n):
    r = lax.broadcasted_iota(jnp.int32, (n, n), 0)
    c = lax.broadcasted_iota(jnp.int32, (n, n), 1)
    return (r == c).astype(jnp.float32)


def _to_row(col, ident):
    return lax.dot_general(
        col, ident, (((0,), (0,)), ((), ())),
        preferred_element_type=jnp.float32,
    )


def _to_col(row, ident):
    return lax.dot_general(
        ident, row, (((1,), (1,)), ((), ())),
        preferred_element_type=jnp.float32,
    )


def kernel(x, gamma, beta):
    m, n = x.shape
    assert m == M
    n_global = n * N_DEV
    g2 = gamma.reshape(1, n)
    b2 = beta.reshape(1, n)

    def body(x_hbm, g_ref, b_ref, o_hbm, xv, ostage, local_stats, commbuf,
             load_sems, store_sems, send_sems, recv_sems):
        my = lax.axis_index("i")

        loads = []
        for c in range(NC):
            cp = pltpu.make_async_copy(
                x_hbm.at[pl.ds(c * TM, TM), :],
                xv.at[pl.ds(c * TM, TM), :],
                load_sems.at[c],
            )
            cp.start()
            loads.append(cp)

        barrier_sem = pltpu.get_barrier_semaphore()
        for p in range(1, N_DEV):
            t = lax.rem(my + p, N_DEV)
            pl.semaphore_signal(
                barrier_sem, inc=1,
                device_id=(t,), device_id_type=pl.DeviceIdType.MESH,
            )
        pl.semaphore_wait(barrier_sem, N_DEV - 1)

        ident_tm = _eye(TM)
        for c in range(NC):
            loads[c].wait()
            xc = xv[pl.ds(c * TM, TM), :]
            s = jnp.sum(xc, axis=1, keepdims=True)
            q = jnp.sum(xc * xc, axis=1, keepdims=True)
            local_stats[0:1, pl.ds(c * TM, TM)] = _to_row(s, ident_tm)
            local_stats[1:2, pl.ds(c * TM, TM)] = _to_row(q, ident_tm)

        rdmas = []
        for p in range(1, N_DEV):
            t = lax.rem(my + p, N_DEV)
            rdma = pltpu.make_async_remote_copy(
                src_ref=local_stats,
                dst_ref=commbuf.at[p - 1],
                send_sem=send_sems.at[p - 1],
                recv_sem=recv_sems.at[p - 1],
                device_id=(t,),
                device_id_type=pl.DeviceIdType.MESH,
            )
            rdma.start()
            rdmas.append(rdma)
        for rdma in rdmas:
            rdma.wait()

        tot = (local_stats[:, :] + commbuf[0] + commbuf[1] + commbuf[2])
        inv_n = 1.0 / n_global
        mean_row = tot[0:1, :] * inv_n
        var_row = tot[1:2, :] * inv_n - mean_row * mean_row
        rstd_row = lax.rsqrt(var_row + EPS)

        g = g_ref[:, :]
        b = b_ref[:, :]
        ident_ts = _eye(TS)
        stores = [None, None]
        for si in range(NS):
            slot = si % 2
            if stores[slot] is not None:
                stores[slot].wait()
            rows = pl.ds(si * TS, TS)
            cols = pl.ds(si * TS, TS)
            mc = _to_col(rstd_row[:, cols] * 0.0 + mean_row[:, cols], ident_ts)
            rc = _to_col(rstd_row[:, cols], ident_ts)
            xc = xv[rows, :]
            ostage[slot] = (xc - mc) * (rc * g) + b
            cp = pltpu.make_async_copy(
                ostage.at[slot],
                o_hbm.at[rows, :],
                store_sems.at[slot],
            )
            cp.start()
            stores[slot] = cp
        for cp in stores:
            if cp is not None:
                cp.wait()

    return pl.pallas_call(
        body,
        out_shape=jax.ShapeDtypeStruct((m, n), jnp.float32),
        in_specs=[
            pl.BlockSpec(memory_space=pltpu.ANY),
            pl.BlockSpec(memory_space=pltpu.VMEM),
            pl.BlockSpec(memory_space=pltpu.VMEM),
        ],
        out_specs=pl.BlockSpec(memory_space=pltpu.ANY),
        scratch_shapes=[
            pltpu.VMEM((M, n), jnp.float32),
            pltpu.VMEM((2, TS, n), jnp.float32),
            pltpu.VMEM((2, M), jnp.float32),
            pltpu.VMEM((N_DEV - 1, 2, M), jnp.float32),
            pltpu.SemaphoreType.DMA((NC,)),
            pltpu.SemaphoreType.DMA((2,)),
            pltpu.SemaphoreType.DMA((N_DEV - 1,)),
            pltpu.SemaphoreType.DMA((N_DEV - 1,)),
        ],
        compiler_params=pltpu.CompilerParams(collective_id=0),
    )(x, g2, b2)


# baseline (device time: 79463 ns/iter reference)
import jax
import jax.numpy as jnp
from jax import lax
from jax.experimental import pallas as pl
from jax.experimental.pallas import tpu as pltpu

N_DEV = 4
M = 6144
EPS = 1e-5

TM = 512
TS = 256
NC = M // TM
NS = M // TS


def _eye(n):
    r = lax.broadcasted_iota(jnp.int32, (n, n), 0)
    c = lax.broadcasted_iota(jnp.int32, (n, n), 1)
    return (r == c).astype(jnp.float32)


def _to_row(col, ident):
    return lax.dot_general(
        col, ident, (((0,), (0,)), ((), ())),
        preferred_element_type=jnp.float32,
    )


def _to_col(row, ident):
    return lax.dot_general(
        ident, row, (((1,), (1,)), ((), ())),
        preferred_element_type=jnp.float32,
    )


def kernel(x, gamma, beta):
    m, n = x.shape
    assert m == M
    n_global = n * N_DEV
    g2 = gamma.reshape(1, n)
    b2 = beta.reshape(1, n)

    def body(x_hbm, g_ref, b_ref, o_hbm, xv, ostage, local_stats, commbuf,
             load_sems, store_sems, send_sems, recv_sems):
        my = lax.axis_index("i")

        loads = []
        for c in range(NC):
            cp = pltpu.make_async_copy(
                x_hbm.at[pl.ds(c * TM, TM), :],
                xv.at[pl.ds(c * TM, TM), :],
                load_sems.at[c],
            )
            cp.start()
            loads.append(cp)

        barrier_sem = pltpu.get_barrier_semaphore()
        for p in range(1, N_DEV):
            t = lax.rem(my + p, N_DEV)
            pl.semaphore_signal(
                barrier_sem, inc=1,
                device_id=(t,), device_id_type=pl.DeviceIdType.MESH,
            )
        pl.semaphore_wait(barrier_sem, N_DEV - 1)

        ident_tm = _eye(TM)
        for c in range(NC):
            loads[c].wait()
            xc = xv[pl.ds(c * TM, TM), :]
            s = jnp.sum(xc, axis=1, keepdims=True)
            q = jnp.sum(xc * xc, axis=1, keepdims=True)
            local_stats[0:1, pl.ds(c * TM, TM)] = _to_row(s, ident_tm)
            local_stats[1:2, pl.ds(c * TM, TM)] = _to_row(q, ident_tm)

        rdmas = []
        for p in range(1, N_DEV):
            t = lax.rem(my + p, N_DEV)
            rdma = pltpu.make_async_remote_copy(
                src_ref=local_stats,
                dst_ref=commbuf.at[p - 1],
                send_sem=send_sems.at[p - 1],
                recv_sem=recv_sems.at[p - 1],
                device_id=(t,),
                device_id_type=pl.DeviceIdType.MESH,
            )
            rdma.start()
            rdmas.append(rdma)
        for rdma in rdmas:
            rdma.wait()

        tot = (local_stats[:, :] + commbuf[0] + commbuf[1] + commbuf[2])
        inv_n = 1.0 / n_global
        mean_row = tot[0:1, :] * inv_n
        var_row = tot[1:2, :] * inv_n - mean_row * mean_row
        rstd_row = lax.rsqrt(var_row + EPS)

        g = g_ref[:, :]
        b = b_ref[:, :]
        ident_ts = _eye(TS)
        stores = [None, None]
        for si in range(NS):
            slot = si % 2
            if stores[slot] is not None:
                stores[slot].wait()
            rows = pl.ds(si * TS, TS)
            mc = _to_col(mean_row[:, si * TS:(si + 1) * TS], ident_ts)
            rc = _to_col(rstd_row[:, si * TS:(si + 1) * TS], ident_ts)
            xc = xv[rows, :]
            ostage[slot] = (xc - mc) * (rc * g) + b
            cp = pltpu.make_async_copy(
                ostage.at[slot],
                o_hbm.at[rows, :],
                store_sems.at[slot],
            )
            cp.start()
            stores[slot] = cp
        for cp in stores:
            if cp is not None:
                cp.wait()

    return pl.pallas_call(
        body,
        out_shape=jax.ShapeDtypeStruct((m, n), jnp.float32),
        in_specs=[
            pl.BlockSpec(memory_space=pl.ANY),
            pl.BlockSpec(memory_space=pltpu.MemorySpace.VMEM),
            pl.BlockSpec(memory_space=pltpu.MemorySpace.VMEM),
        ],
        out_specs=pl.BlockSpec(memory_space=pl.ANY),
        scratch_shapes=[
            pltpu.MemorySpace.VMEM((M, n), jnp.float32),
            pltpu.MemorySpace.VMEM((2, TS, n), jnp.float32),
            pltpu.MemorySpace.VMEM((2, M), jnp.float32),
            pltpu.MemorySpace.VMEM((N_DEV - 1, 2, M), jnp.float32),
            pltpu.SemaphoreType.DMA((NC,)),
            pltpu.SemaphoreType.DMA((2,)),
            pltpu.SemaphoreType.DMA((N_DEV - 1,)),
            pltpu.SemaphoreType.DMA((N_DEV - 1,)),
        ],
        compiler_params=pltpu.CompilerParams(
            collective_id=0,
            vmem_limit_bytes=64 * 1024 * 1024,
        ),
    )(x, g2, b2)


# device time: 57519 ns/iter; 1.3815x vs baseline; 1.3815x over previous
import jax
import jax.numpy as jnp
from jax import lax
from jax.experimental import pallas as pl
from jax.experimental.pallas import tpu as pltpu

N_DEV = 4
M = 6144
EPS = 1e-5

TM = 512
NC = M // TM
TS = 512
NS = M // TS

_VMEM = pltpu.MemorySpace.VMEM


def _eye(n):
    r = lax.broadcasted_iota(jnp.int32, (n, n), 0)
    c = lax.broadcasted_iota(jnp.int32, (n, n), 1)
    return (r == c).astype(jnp.float32)


def _to_row(col, ident):
    return lax.dot_general(
        col, ident, (((0,), (0,)), ((), ())),
        preferred_element_type=jnp.float32,
    )


def _to_col(row, ident):
    return lax.dot_general(
        ident, row, (((1,), (1,)), ((), ())),
        preferred_element_type=jnp.float32,
    )


def _stats_allreduce(x):
    m, n = x.shape

    def body(x_hbm, o_ref, xv, local_stats, commbuf,
             load_sems, send_sems, recv_sems):
        my = lax.axis_index("i")

        loads = []
        for c in range(NC):
            cp = pltpu.make_async_copy(
                x_hbm.at[pl.ds(c * TM, TM), :],
                xv.at[pl.ds(c * TM, TM), :],
                load_sems.at[c],
            )
            cp.start()
            loads.append(cp)

        barrier_sem = pltpu.get_barrier_semaphore()
        for p in range(1, N_DEV):
            t = lax.rem(my + p, N_DEV)
            pl.semaphore_signal(
                barrier_sem, inc=1,
                device_id=(t,), device_id_type=pl.DeviceIdType.MESH,
            )
        pl.semaphore_wait(barrier_sem, N_DEV - 1)

        ident_tm = _eye(TM)
        for c in range(NC):
            loads[c].wait()
            xc = xv[pl.ds(c * TM, TM), :]
            s = jnp.sum(xc, axis=1, keepdims=True)
            q = jnp.sum(xc * xc, axis=1, keepdims=True)
            local_stats[0:1, pl.ds(c * TM, TM)] = _to_row(s, ident_tm)
            local_stats[1:2, pl.ds(c * TM, TM)] = _to_row(q, ident_tm)

        rdmas = []
        for p in range(1, N_DEV):
            t = lax.rem(my + p, N_DEV)
            rdma = pltpu.make_async_remote_copy(
                src_ref=local_stats,
                dst_ref=commbuf.at[p - 1],
                send_sem=send_sems.at[p - 1],
                recv_sem=recv_sems.at[p - 1],
                device_id=(t,),
                device_id_type=pl.DeviceIdType.MESH,
            )
            rdma.start()
            rdmas.append(rdma)
        for rdma in rdmas:
            rdma.wait()

        o_ref[:, :] = (local_stats[:, :] + commbuf[0] + commbuf[1]
                       + commbuf[2])

    return pl.pallas_call(
        body,
        out_shape=jax.ShapeDtypeStruct((2, M), jnp.float32),
        in_specs=[pl.BlockSpec(memory_space=pl.ANY)],
        out_specs=pl.BlockSpec(memory_space=_VMEM),
        scratch_shapes=[
            _VMEM((M, n), jnp.float32),
            _VMEM((2, M), jnp.float32),
            _VMEM((N_DEV - 1, 2, M), jnp.float32),
            pltpu.SemaphoreType.DMA((NC,)),
            pltpu.SemaphoreType.DMA((N_DEV - 1,)),
            pltpu.SemaphoreType.DMA((N_DEV - 1,)),
        ],
        compiler_params=pltpu.CompilerParams(
            collective_id=0,
            vmem_limit_bytes=64 * 1024 * 1024,
        ),
    )(x)


def _normalize(x, tot, g2, b2):
    m, n = x.shape
    n_global = n * N_DEV

    def body(x_hbm, tot_ref, g_ref, b_ref, o_hbm, xbuf, obuf,
             load_sems, store_sems):
        inv_n = 1.0 / n_global
        mean_row = tot_ref[0:1, :] * inv_n
        var_row = tot_ref[1:2, :] * inv_n - mean_row * mean_row
        rstd_row = lax.rsqrt(var_row + EPS)
        g = g_ref[:, :]
        b = b_ref[:, :]
        ident_ts = _eye(TS)

        loads = [None] * NS
        stores = [None] * NS
        for c in range(min(2, NS)):
            cp = pltpu.make_async_copy(
                x_hbm.at[pl.ds(c * TS, TS), :],
                xbuf.at[c % 2],
                load_sems.at[c % 2],
            )
            cp.start()
            loads[c] = cp

        for si in range(NS):
            slot = si % 2
            loads[si].wait()
            if si >= 2:
                stores[si - 2].wait()
            mc = _to_col(mean_row[:, si * TS:(si + 1) * TS], ident_ts)
            rc = _to_col(rstd_row[:, si * TS:(si + 1) * TS], ident_ts)
            obuf[slot] = (xbuf[slot] - mc) * (rc * g) + b
            st = pltpu.make_async_copy(
                obuf.at[slot],
                o_hbm.at[pl.ds(si * TS, TS), :],
                store_sems.at[slot],
            )
            st.start()
            stores[si] = st
            if si + 2 < NS:
                cp = pltpu.make_async_copy(
                    x_hbm.at[pl.ds((si + 2) * TS, TS), :],
                    xbuf.at[slot],
                    load_sems.at[slot],
                )
                cp.start()
                loads[si + 2] = cp
        for si in range(max(NS - 2, 0), NS):
            stores[si].wait()

    return pl.pallas_call(
        body,
        out_shape=jax.ShapeDtypeStruct((m, n), jnp.float32),
        in_specs=[
            pl.BlockSpec(memory_space=pl.ANY),
            pl.BlockSpec(memory_space=_VMEM),
            pl.BlockSpec(memory_space=_VMEM),
            pl.BlockSpec(memory_space=_VMEM),
        ],
        out_specs=pl.BlockSpec(memory_space=pl.ANY),
        scratch_shapes=[
            _VMEM((2, TS, n), jnp.float32),
            _VMEM((2, TS, n), jnp.float32),
            pltpu.SemaphoreType.DMA((2,)),
            pltpu.SemaphoreType.DMA((2,)),
        ],
        compiler_params=pltpu.CompilerParams(
            vmem_limit_bytes=64 * 1024 * 1024,
        ),
    )(x, tot, g2, b2)


def kernel(x, gamma, beta):
    n = x.shape[1]
    g2 = gamma.reshape(1, n)
    b2 = beta.reshape(1, n)
    tot = _stats_allreduce(x)
    return _normalize(x, tot, g2, b2)


# device time: 56268 ns/iter; 1.4122x vs baseline; 1.0222x over previous
import jax
import jax.numpy as jnp
from jax import lax
from jax.experimental import pallas as pl
from jax.experimental.pallas import tpu as pltpu

N_DEV = 4
M = 6144
EPS = 1e-5

TM = 512
NC = M // TM
TS = 1024
NS = M // TS

_VMEM = pltpu.MemorySpace.VMEM


def _eye(n):
    r = lax.broadcasted_iota(jnp.int32, (n, n), 0)
    c = lax.broadcasted_iota(jnp.int32, (n, n), 1)
    return (r == c).astype(jnp.float32)


def _to_row(col, ident):
    return lax.dot_general(
        col, ident, (((0,), (0,)), ((), ())),
        preferred_element_type=jnp.float32,
    )


def _to_col(row, ident):
    return lax.dot_general(
        ident, row, (((1,), (1,)), ((), ())),
        preferred_element_type=jnp.float32,
    )


def _stats_allreduce(x):
    m, n = x.shape

    def body(x_hbm, o_ref, xv, local_stats, commbuf,
             load_sems, send_sems, recv_sems):
        my = lax.axis_index("i")

        loads = []
        for c in range(NC):
            cp = pltpu.make_async_copy(
                x_hbm.at[pl.ds(c * TM, TM), :],
                xv.at[pl.ds(c * TM, TM), :],
                load_sems.at[c],
            )
            cp.start()
            loads.append(cp)

        barrier_sem = pltpu.get_barrier_semaphore()
        for p in range(1, N_DEV):
            t = lax.rem(my + p, N_DEV)
            pl.semaphore_signal(
                barrier_sem, inc=1,
                device_id=(t,), device_id_type=pl.DeviceIdType.MESH,
            )
        pl.semaphore_wait(barrier_sem, N_DEV - 1)

        ident_tm = _eye(TM)
        for c in range(NC):
            loads[c].wait()
            xc = xv[pl.ds(c * TM, TM), :]
            s = jnp.sum(xc, axis=1, keepdims=True)
            q = jnp.sum(xc * xc, axis=1, keepdims=True)
            local_stats[0:1, pl.ds(c * TM, TM)] = _to_row(s, ident_tm)
            local_stats[1:2, pl.ds(c * TM, TM)] = _to_row(q, ident_tm)

        rdmas = []
        for p in range(1, N_DEV):
            t = lax.rem(my + p, N_DEV)
            rdma = pltpu.make_async_remote_copy(
                src_ref=local_stats,
                dst_ref=commbuf.at[p - 1],
                send_sem=send_sems.at[p - 1],
                recv_sem=recv_sems.at[p - 1],
                device_id=(t,),
                device_id_type=pl.DeviceIdType.MESH,
            )
            rdma.start()
            rdmas.append(rdma)
        for rdma in rdmas:
            rdma.wait()

        o_ref[:, :] = (local_stats[:, :] + commbuf[0] + commbuf[1]
                       + commbuf[2])

    return pl.pallas_call(
        body,
        out_shape=jax.ShapeDtypeStruct((2, M), jnp.float32),
        in_specs=[pl.BlockSpec(memory_space=pl.ANY)],
        out_specs=pl.BlockSpec(memory_space=_VMEM),
        scratch_shapes=[
            _VMEM((M, n), jnp.float32),
            _VMEM((2, M), jnp.float32),
            _VMEM((N_DEV - 1, 2, M), jnp.float32),
            pltpu.SemaphoreType.DMA((NC,)),
            pltpu.SemaphoreType.DMA((N_DEV - 1,)),
            pltpu.SemaphoreType.DMA((N_DEV - 1,)),
        ],
        compiler_params=pltpu.CompilerParams(
            collective_id=0,
            vmem_limit_bytes=64 * 1024 * 1024,
        ),
    )(x)


def _normalize(x, tot, g2, b2):
    m, n = x.shape
    n_global = n * N_DEV

    def body(x_hbm, tot_ref, g_ref, b_ref, o_hbm, xbuf, obuf,
             load_sems, store_sems):
        inv_n = 1.0 / n_global
        mean_row = tot_ref[0:1, :] * inv_n
        var_row = tot_ref[1:2, :] * inv_n - mean_row * mean_row
        rstd_row = lax.rsqrt(var_row + EPS)
        g = g_ref[:, :]
        b = b_ref[:, :]
        ident_ts = _eye(TS)

        loads = [None] * NS
        stores = [None] * NS
        for c in range(min(2, NS)):
            cp = pltpu.make_async_copy(
                x_hbm.at[pl.ds(c * TS, TS), :],
                xbuf.at[c % 2],
                load_sems.at[c % 2],
            )
            cp.start()
            loads[c] = cp

        for si in range(NS):
            slot = si % 2
            loads[si].wait()
            if si >= 2:
                stores[si - 2].wait()
            mc = _to_col(mean_row[:, si * TS:(si + 1) * TS], ident_ts)
            rc = _to_col(rstd_row[:, si * TS:(si + 1) * TS], ident_ts)
            obuf[slot] = (xbuf[slot] - mc) * (rc * g) + b
            st = pltpu.make_async_copy(
                obuf.at[slot],
                o_hbm.at[pl.ds(si * TS, TS), :],
                store_sems.at[slot],
            )
            st.start()
            stores[si] = st
            if si + 2 < NS:
                cp = pltpu.make_async_copy(
                    x_hbm.at[pl.ds((si + 2) * TS, TS), :],
                    xbuf.at[slot],
                    load_sems.at[slot],
                )
                cp.start()
                loads[si + 2] = cp
        for si in range(max(NS - 2, 0), NS):
            stores[si].wait()

    return pl.pallas_call(
        body,
        out_shape=jax.ShapeDtypeStruct((m, n), jnp.float32),
        in_specs=[
            pl.BlockSpec(memory_space=pl.ANY),
            pl.BlockSpec(memory_space=_VMEM),
            pl.BlockSpec(memory_space=_VMEM),
            pl.BlockSpec(memory_space=_VMEM),
        ],
        out_specs=pl.BlockSpec(memory_space=pl.ANY),
        scratch_shapes=[
            _VMEM((2, TS, n), jnp.float32),
            _VMEM((2, TS, n), jnp.float32),
            pltpu.SemaphoreType.DMA((2,)),
            pltpu.SemaphoreType.DMA((2,)),
        ],
        compiler_params=pltpu.CompilerParams(
            vmem_limit_bytes=64 * 1024 * 1024,
        ),
    )(x, tot, g2, b2)


def kernel(x, gamma, beta):
    n = x.shape[1]
    g2 = gamma.reshape(1, n)
    b2 = beta.reshape(1, n)
    tot = _stats_allreduce(x)
    return _normalize(x, tot, g2, b2)
